# baseline (speedup 1.0000x reference)
import jax
import jax.numpy as jnp
from jax.experimental import pallas as pl
from jax.experimental.pallas import tpu as pltpu

NF = 384
B = 32
HW = 1024
CH = 128
NCH = NF // CH


def _affine_body(y_ref, e_ref, x_ref, o_ref):
    j = pl.program_id(1)
    off = pl.multiple_of(j * CH, 128)
    g = e_ref[0, 0, pl.ds(off, CH)].reshape(CH, 1)
    b = e_ref[0, 0, pl.ds(NF + off, CH)].reshape(CH, 1)
    o_ref[0] = x_ref[0] * g + b


def kernel(x, y, embed_weight):
    xr = x.reshape(B, NF, HW)
    y32 = y.astype(jnp.int32)
    e3 = embed_weight.reshape(-1, 1, 2 * NF)
    grid_spec = pltpu.PrefetchScalarGridSpec(
        num_scalar_prefetch=1,
        grid=(B, NCH),
        in_specs=[
            pl.BlockSpec((1, 1, 2 * NF), lambda bi, j, yv: (yv[bi], 0, 0)),
            pl.BlockSpec((1, CH, HW), lambda bi, j, yv: (bi, j, 0)),
        ],
        out_specs=pl.BlockSpec((1, CH, HW), lambda bi, j, yv: (bi, j, 0)),
    )
    out = pl.pallas_call(
        _affine_body,
        grid_spec=grid_spec,
        out_shape=jax.ShapeDtypeStruct((B, NF, HW), jnp.float32),
    )(y32, e3, xr)
    return out.reshape(x.shape)
